# baseline (speedup 1.0000x reference)
import functools

import jax
import jax.numpy as jnp
from jax import lax
from jax.experimental import pallas as pl
from jax.experimental.pallas import tpu as pltpu
from jax.experimental.pallas import tpu_sc as plsc

TOPK = 30
NSLICE = 16
NCAND = 5
NEG_INF = float("-inf")


def _oddeven_merge(lo, n, r):
    step = r * 2
    if step < n:
        yield from _oddeven_merge(lo, n, step)
        yield from _oddeven_merge(lo + r, n, step)
        for i in range(lo + r, lo + n - r, step):
            yield (i, i + r)
    else:
        yield (lo, lo + r)


def _oddeven_merge_sort(lo, hi):
    if hi - lo >= 1:
        mid = lo + (hi - lo) // 2
        yield from _oddeven_merge_sort(lo, mid)
        yield from _oddeven_merge_sort(mid + 1, hi)
        yield from _oddeven_merge(lo, hi - lo + 1, 1)


_SORT_PAIRS = list(_oddeven_merge_sort(0, NSLICE - 1))


def _extract_kth_max(arr, m, n_pulls):
    def step(_, carry):
        cur, t = carry
        mi = jnp.max(cur, axis=1, keepdims=True)
        cur = jnp.where(cur >= mi, NEG_INF, cur)
        return cur, mi

    _, t = lax.fori_loop(0, n_pulls, step, (arr, m))
    return t


def _scores_kernel(q_ref, k_ref, s_ref):
    s_ref[0] = jax.lax.dot_general(
        q_ref[0], k_ref[0], (((1,), (1,)), ((), ())),
        preferred_element_type=jnp.float32,
    )


def _sc_rowmax(scores2d):
    R, S = scores2d.shape
    info = plsc.get_sparse_core_info()
    nw = info.num_cores * info.num_subcores
    rpw = R // nw
    nch = S // info.num_lanes
    mesh = plsc.VectorSubcoreMesh(core_axis_name="c", subcore_axis_name="s")

    @functools.partial(
        pl.kernel,
        mesh=mesh,
        out_type=jax.ShapeDtypeStruct((R,), jnp.float32),
        scratch_types=[
            pltpu.VMEM((2, S), jnp.float32),
            pltpu.VMEM((rpw,), jnp.float32),
            pltpu.SemaphoreType.DMA,
            pltpu.SemaphoreType.DMA,
        ],
    )
    def sc_k(s_hbm, out_hbm, rowbuf, outbuf, sem0, sem1):
        wid = lax.axis_index("s") * info.num_cores + lax.axis_index("c")
        base = wid * rpw

        lanes = lax.broadcasted_iota(jnp.int32, (16,), 0)

        def xlane_max(r):
            for sh in (1, 2, 4, 8):
                r = jnp.maximum(r, r[lanes ^ sh])
            return r

        def row_max(b, i, acc):
            def ch(j, r):
                return jnp.maximum(r, rowbuf.at[b][pl.ds(j * 16, 16)][...])

            r = lax.fori_loop(
                0, nch, ch, jnp.full((16,), NEG_INF, jnp.float32), unroll=8
            )
            return jnp.where(lanes == (i % 16), xlane_max(r), acc)

        pltpu.make_async_copy(s_hbm.at[base], rowbuf.at[0], sem0).start()

        def body(g, acc):
            i0 = 2 * g
            pltpu.make_async_copy(
                s_hbm.at[base + i0 + 1], rowbuf.at[1], sem1
            ).start()
            pltpu.make_async_copy(s_hbm.at[base + i0], rowbuf.at[0], sem0).wait()
            acc = row_max(0, i0, acc)

            @pl.when(i0 + 2 < rpw)
            def _():
                pltpu.make_async_copy(
                    s_hbm.at[base + i0 + 2], rowbuf.at[0], sem0
                ).start()

            pltpu.make_async_copy(
                s_hbm.at[base + i0 + 1], rowbuf.at[1], sem1
            ).wait()
            acc = row_max(1, i0 + 1, acc)

            @pl.when(g % 8 == 7)
            def _():
                outbuf[pl.ds((g // 8) * 16, 16)] = acc

            return acc

        lax.fori_loop(0, rpw // 2, body, jnp.zeros((16,), jnp.float32))
        pltpu.sync_copy(outbuf, out_hbm.at[pl.ds(base, rpw)])

    return sc_k(scores2d)


def _attn_block_kernel(s_ref, m_ref, v_ref, attn_ref, ctx_ref, t_ref):
    s = s_ref[0]
    S = s.shape[1]
    w = S // NSLICE
    m = m_ref[0, 0].reshape(-1, 1)

    sl = [s[:, i * w : (i + 1) * w] for i in range(NSLICE)]
    for i, j in _SORT_PAIRS:
        hi = jnp.maximum(sl[i], sl[j])
        lo = jnp.minimum(sl[i], sl[j])
        sl[i], sl[j] = hi, lo

    cand = jnp.concatenate(sl[:NCAND], axis=1)
    t_cand = _extract_kth_max(cand, m, TOPK)

    c_gt = jnp.sum((s > t_cand).astype(jnp.float32), axis=1, keepdims=True)
    c_ge = jnp.sum((s >= t_cand).astype(jnp.float32), axis=1, keepdims=True)
    ok = jnp.logical_and(c_gt < TOPK, c_ge >= TOPK)
    t_ref[...] = t_cand

    @pl.when(jnp.logical_not(jnp.all(ok)))
    def _fallback():
        t_ref[...] = _extract_kth_max(s, m, TOPK)

    t = t_ref[...]
    wexp = jnp.where(s >= t, jnp.exp(s - m), 0.0)
    z = jnp.sum(wexp, axis=1, keepdims=True)
    attn = wexp / z
    attn_ref[0] = attn
    ctx_ref[0] = jax.lax.dot_general(
        attn, v_ref[0], (((1,), (0,)), ((), ())), preferred_element_type=jnp.float32
    )


@jax.jit
def _run(q, k, v):
    bh, S, d = q.shape
    blk = min(256, S)
    grid = (bh, S // blk)

    scores = pl.pallas_call(
        _scores_kernel,
        grid=grid,
        in_specs=[
            pl.BlockSpec((1, blk, d), lambda h, i: (h, i, 0)),
            pl.BlockSpec((1, S, d), lambda h, i: (h, 0, 0)),
        ],
        out_specs=pl.BlockSpec((1, blk, S), lambda h, i: (h, i, 0)),
        out_shape=jax.ShapeDtypeStruct((bh, S, S), jnp.float32),
        compiler_params=pltpu.CompilerParams(
            dimension_semantics=("parallel", "arbitrary"),
        ),
    )(q, k)

    m = _sc_rowmax(scores.reshape(bh * S, S))
    m4 = m.reshape(bh, S // blk, 1, blk)

    attn, ctx = pl.pallas_call(
        _attn_block_kernel,
        grid=grid,
        in_specs=[
            pl.BlockSpec((1, blk, S), lambda h, i: (h, i, 0)),
            pl.BlockSpec((1, 1, 1, blk), lambda h, i: (h, i, 0, 0)),
            pl.BlockSpec((1, S, d), lambda h, i: (h, 0, 0)),
        ],
        out_specs=[
            pl.BlockSpec((1, blk, S), lambda h, i: (h, i, 0)),
            pl.BlockSpec((1, blk, d), lambda h, i: (h, i, 0)),
        ],
        out_shape=[
            jax.ShapeDtypeStruct((bh, S, S), jnp.float32),
            jax.ShapeDtypeStruct((bh, S, d), jnp.float32),
        ],
        scratch_shapes=[pltpu.VMEM((blk, 1), jnp.float32)],
        compiler_params=pltpu.CompilerParams(
            dimension_semantics=("parallel", "arbitrary"),
        ),
    )(scores, m4, v)
    return ctx, attn


def kernel(q, k, v, B, num_heads):
    return _run(q, k, v)


# speedup vs baseline: 2.0701x; 2.0701x over previous
import functools

import jax
import jax.numpy as jnp
from jax.experimental import pallas as pl
from jax.experimental.pallas import tpu as pltpu

TOPK = 30
NSLICE = 16
NCAND = 5
NEG_INF = float("-inf")


def _oddeven_merge(lo, n, r):
    step = r * 2
    if step < n:
        yield from _oddeven_merge(lo, n, step)
        yield from _oddeven_merge(lo + r, n, step)
        for i in range(lo + r, lo + n - r, step):
            yield (i, i + r)
    else:
        yield (lo, lo + r)


def _oddeven_merge_sort(lo, hi):
    if hi - lo >= 1:
        mid = lo + (hi - lo) // 2
        yield from _oddeven_merge_sort(lo, mid)
        yield from _oddeven_merge_sort(mid + 1, hi)
        yield from _oddeven_merge(lo, hi - lo + 1, 1)


def _prune_for_top(pairs, n_top):
    needed = set(range(n_top))
    kept = []
    for i, j in reversed(pairs):
        if i in needed or j in needed:
            kept.append((i, j))
            needed.add(i)
            needed.add(j)
    return list(reversed(kept))


_SORT_PAIRS = _prune_for_top(list(_oddeven_merge_sort(0, NSLICE - 1)), NCAND)


def _extract_kth_max(arr, m, n_pulls):
    def step(_, carry):
        cur, t = carry
        mi = jnp.max(cur, axis=1, keepdims=True)
        cur = jnp.where(cur >= mi, NEG_INF, cur)
        return cur, mi

    _, t = jax.lax.fori_loop(0, n_pulls, step, (arr, m), unroll=5)
    return t


def _attn_block_kernel(q_ref, k_ref, v_ref, attn_ref, ctx_ref, t_ref):
    qb = q_ref[0]
    kb = k_ref[0]
    s = jax.lax.dot_general(
        qb, kb, (((1,), (1,)), ((), ())), preferred_element_type=jnp.float32
    )
    S = s.shape[1]
    w = S // NSLICE

    m = jnp.max(s, axis=1, keepdims=True)

    sl = [s[:, i * w : (i + 1) * w] for i in range(NSLICE)]
    for i, j in _SORT_PAIRS:
        hi = jnp.maximum(sl[i], sl[j])
        lo = jnp.minimum(sl[i], sl[j])
        sl[i], sl[j] = hi, lo

    cand = jnp.concatenate(sl[:NCAND], axis=1)
    t_cand = _extract_kth_max(cand, m, TOPK)

    c_gt = jnp.sum((s > t_cand).astype(jnp.float32), axis=1, keepdims=True)
    ok = c_gt < TOPK
    t_ref[...] = t_cand

    @pl.when(jnp.logical_not(jnp.all(ok)))
    def _fallback():
        t_ref[...] = _extract_kth_max(s, m, TOPK)

    t = t_ref[...]
    wexp = jnp.where(s >= t, jnp.exp(s - m), 0.0)
    z = jnp.sum(wexp, axis=1, keepdims=True)
    attn = wexp / z
    attn_ref[0] = attn
    ctx_ref[0] = jax.lax.dot_general(
        attn, v_ref[0], (((1,), (0,)), ((), ())), preferred_element_type=jnp.float32
    )


@functools.partial(jax.jit, static_argnames=("interpret",))
def _run(q, k, v, interpret=False):
    bh, S, d = q.shape
    blk = min(256, S)
    grid = (bh, S // blk)
    attn, ctx = pl.pallas_call(
        _attn_block_kernel,
        grid=grid,
        in_specs=[
            pl.BlockSpec((1, blk, d), lambda h, i: (h, i, 0)),
            pl.BlockSpec((1, S, d), lambda h, i: (h, 0, 0)),
            pl.BlockSpec((1, S, d), lambda h, i: (h, 0, 0)),
        ],
        out_specs=[
            pl.BlockSpec((1, blk, S), lambda h, i: (h, i, 0)),
            pl.BlockSpec((1, blk, d), lambda h, i: (h, i, 0)),
        ],
        out_shape=[
            jax.ShapeDtypeStruct((bh, S, S), jnp.float32),
            jax.ShapeDtypeStruct((bh, S, d), jnp.float32),
        ],
        scratch_shapes=[pltpu.VMEM((blk, 1), jnp.float32)],
        compiler_params=pltpu.CompilerParams(
            dimension_semantics=("parallel", "arbitrary"),
        ),
        interpret=interpret,
    )(q, k, v)
    return ctx, attn


def kernel(q, k, v, B, num_heads):
    return _run(q, k, v)


# speedup vs baseline: 2.2147x; 1.0699x over previous
import functools

import jax
import jax.numpy as jnp
from jax.experimental import pallas as pl
from jax.experimental.pallas import tpu as pltpu

TOPK = 30
NSLICE = 16
NCAND = 5
NEG_INF = float("-inf")


def _oddeven_merge(lo, n, r):
    step = r * 2
    if step < n:
        yield from _oddeven_merge(lo, n, step)
        yield from _oddeven_merge(lo + r, n, step)
        for i in range(lo + r, lo + n - r, step):
            yield (i, i + r)
    else:
        yield (lo, lo + r)


def _oddeven_merge_sort(lo, hi):
    if hi - lo >= 1:
        mid = lo + (hi - lo) // 2
        yield from _oddeven_merge_sort(lo, mid)
        yield from _oddeven_merge_sort(mid + 1, hi)
        yield from _oddeven_merge(lo, hi - lo + 1, 1)


def _prune_for_top(pairs, n_top):
    needed = set(range(n_top))
    kept = []
    for i, j in reversed(pairs):
        if i in needed or j in needed:
            kept.append((i, j))
            needed.add(i)
            needed.add(j)
    return list(reversed(kept))


_SORT_PAIRS = _prune_for_top(list(_oddeven_merge_sort(0, NSLICE - 1)), NCAND)


def _extract_kth_max(arr, m, n_pulls):
    def step(_, carry):
        cur, t = carry
        mi = jnp.max(cur, axis=1, keepdims=True)
        cur = jnp.where(cur >= mi, NEG_INF, cur)
        return cur, mi

    _, t = jax.lax.fori_loop(0, n_pulls, step, (arr, m), unroll=10)
    return t


def _attn_block_kernel(q_ref, k_ref, v_ref, attn_ref, ctx_ref, t_ref):
    qb = q_ref[0]
    kb = k_ref[0]
    s = jax.lax.dot_general(
        qb, kb, (((1,), (1,)), ((), ())), preferred_element_type=jnp.float32
    )
    S = s.shape[1]
    w = S // NSLICE

    m = jnp.max(s, axis=1, keepdims=True)

    sl = [s[:, i * w : (i + 1) * w] for i in range(NSLICE)]
    for i, j in _SORT_PAIRS:
        hi = jnp.maximum(sl[i], sl[j])
        lo = jnp.minimum(sl[i], sl[j])
        sl[i], sl[j] = hi, lo

    cand = jnp.concatenate(sl[:NCAND], axis=1)
    t_cand = _extract_kth_max(cand, m, TOPK)

    c_gt = jnp.sum((s > t_cand).astype(jnp.float32), axis=1, keepdims=True)
    ok = c_gt < TOPK
    t_ref[...] = t_cand

    @pl.when(jnp.logical_not(jnp.all(ok)))
    def _fallback():
        t_ref[...] = _extract_kth_max(s, m, TOPK)

    t = t_ref[...]
    wexp = jnp.where(s >= t, jnp.exp(s - m), 0.0)
    z = jnp.sum(wexp, axis=1, keepdims=True)
    attn = wexp / z
    attn_ref[0] = attn
    ctx_ref[0] = jax.lax.dot_general(
        attn, v_ref[0], (((1,), (0,)), ((), ())), preferred_element_type=jnp.float32
    )


@functools.partial(jax.jit, static_argnames=("interpret",))
def _run(q, k, v, interpret=False):
    bh, S, d = q.shape
    blk = min(256, S)
    grid = (bh, S // blk)
    attn, ctx = pl.pallas_call(
        _attn_block_kernel,
        grid=grid,
        in_specs=[
            pl.BlockSpec((1, blk, d), lambda h, i: (h, i, 0)),
            pl.BlockSpec((1, S, d), lambda h, i: (h, 0, 0)),
            pl.BlockSpec((1, S, d), lambda h, i: (h, 0, 0)),
        ],
        out_specs=[
            pl.BlockSpec((1, blk, S), lambda h, i: (h, i, 0)),
            pl.BlockSpec((1, blk, d), lambda h, i: (h, i, 0)),
        ],
        out_shape=[
            jax.ShapeDtypeStruct((bh, S, S), jnp.float32),
            jax.ShapeDtypeStruct((bh, S, d), jnp.float32),
        ],
        scratch_shapes=[pltpu.VMEM((blk, 1), jnp.float32)],
        compiler_params=pltpu.CompilerParams(
            dimension_semantics=("parallel", "arbitrary"),
        ),
        interpret=interpret,
    )(q, k, v)
    return ctx, attn


def kernel(q, k, v, B, num_heads):
    return _run(q, k, v)


# speedup vs baseline: 2.4795x; 1.1195x over previous
import functools

import jax
import jax.numpy as jnp
from jax.experimental import pallas as pl
from jax.experimental.pallas import tpu as pltpu

TOPK = 30
NSLICE = 16
NCAND = 5
NEG_INF = float("-inf")


def _oddeven_merge(lo, n, r):
    step = r * 2
    if step < n:
        yield from _oddeven_merge(lo, n, step)
        yield from _oddeven_merge(lo + r, n, step)
        for i in range(lo + r, lo + n - r, step):
            yield (i, i + r)
    else:
        yield (lo, lo + r)


def _oddeven_merge_sort(lo, hi):
    if hi - lo >= 1:
        mid = lo + (hi - lo) // 2
        yield from _oddeven_merge_sort(lo, mid)
        yield from _oddeven_merge_sort(mid + 1, hi)
        yield from _oddeven_merge(lo, hi - lo + 1, 1)


def _prune_for_top(pairs, n_top):
    needed = set(range(n_top))
    kept = []
    for i, j in reversed(pairs):
        if i in needed or j in needed:
            kept.append((i, j))
            needed.add(i)
            needed.add(j)
    return list(reversed(kept))


_SORT_PAIRS = _prune_for_top(list(_oddeven_merge_sort(0, NSLICE - 1)), NCAND)


def _extract_kth_max(arr, m, n_pulls):
    def step(_, carry):
        cur, t = carry
        mi = jnp.max(cur, axis=1, keepdims=True)
        cur = jnp.where(cur >= mi, NEG_INF, cur)
        return cur, mi

    _, t = jax.lax.fori_loop(0, n_pulls, step, (arr, m), unroll=30)
    return t


def _attn_block_kernel(q_ref, k_ref, v_ref, attn_ref, ctx_ref, t_ref):
    qb = q_ref[0]
    kb = k_ref[0]
    s = jax.lax.dot_general(
        qb, kb, (((1,), (1,)), ((), ())), preferred_element_type=jnp.float32
    )
    S = s.shape[1]
    w = S // NSLICE

    m = jnp.max(s, axis=1, keepdims=True)

    sl = [s[:, i * w : (i + 1) * w] for i in range(NSLICE)]
    for i, j in _SORT_PAIRS:
        hi = jnp.maximum(sl[i], sl[j])
        lo = jnp.minimum(sl[i], sl[j])
        sl[i], sl[j] = hi, lo

    cand = jnp.concatenate(sl[:NCAND], axis=1)
    t_cand = _extract_kth_max(cand, m, TOPK)

    c_gt = jnp.sum((s > t_cand).astype(jnp.float32), axis=1, keepdims=True)
    ok = c_gt < TOPK
    t_ref[...] = t_cand

    @pl.when(jnp.logical_not(jnp.all(ok)))
    def _fallback():
        t_ref[...] = _extract_kth_max(s, m, TOPK)

    t = t_ref[...]
    wexp = jnp.where(s >= t, jnp.exp(s - m), 0.0)
    z = jnp.sum(wexp, axis=1, keepdims=True)
    attn = wexp / z
    attn_ref[0] = attn
    ctx_ref[0] = jax.lax.dot_general(
        attn, v_ref[0], (((1,), (0,)), ((), ())), preferred_element_type=jnp.float32
    )


@functools.partial(jax.jit, static_argnames=("interpret",))
def _run(q, k, v, interpret=False):
    bh, S, d = q.shape
    blk = min(256, S)
    grid = (bh, S // blk)
    attn, ctx = pl.pallas_call(
        _attn_block_kernel,
        grid=grid,
        in_specs=[
            pl.BlockSpec((1, blk, d), lambda h, i: (h, i, 0)),
            pl.BlockSpec((1, S, d), lambda h, i: (h, 0, 0)),
            pl.BlockSpec((1, S, d), lambda h, i: (h, 0, 0)),
        ],
        out_specs=[
            pl.BlockSpec((1, blk, S), lambda h, i: (h, i, 0)),
            pl.BlockSpec((1, blk, d), lambda h, i: (h, i, 0)),
        ],
        out_shape=[
            jax.ShapeDtypeStruct((bh, S, S), jnp.float32),
            jax.ShapeDtypeStruct((bh, S, d), jnp.float32),
        ],
        scratch_shapes=[pltpu.VMEM((blk, 1), jnp.float32)],
        compiler_params=pltpu.CompilerParams(
            dimension_semantics=("parallel", "arbitrary"),
        ),
        interpret=interpret,
    )(q, k, v)
    return ctx, attn


def kernel(q, k, v, B, num_heads):
    return _run(q, k, v)


# speedup vs baseline: 2.7416x; 1.1057x over previous
import functools

import jax
import jax.numpy as jnp
from jax.experimental import pallas as pl
from jax.experimental.pallas import tpu as pltpu

TOPK = 30
NSLICE = 16
NCAND = 5
NEG_INF = float("-inf")


def _oddeven_merge(lo, n, r):
    step = r * 2
    if step < n:
        yield from _oddeven_merge(lo, n, step)
        yield from _oddeven_merge(lo + r, n, step)
        for i in range(lo + r, lo + n - r, step):
            yield (i, i + r)
    else:
        yield (lo, lo + r)


def _oddeven_merge_sort(lo, hi):
    if hi - lo >= 1:
        mid = lo + (hi - lo) // 2
        yield from _oddeven_merge_sort(lo, mid)
        yield from _oddeven_merge_sort(mid + 1, hi)
        yield from _oddeven_merge(lo, hi - lo + 1, 1)


def _prune_for_top(pairs, n_top):
    needed = set(range(n_top))
    kept = []
    for i, j in reversed(pairs):
        if i in needed or j in needed:
            kept.append((i, j))
            needed.add(i)
            needed.add(j)
    return list(reversed(kept))


_SORT_PAIRS = _prune_for_top(list(_oddeven_merge_sort(0, NSLICE - 1)), NCAND)


def _extract_kth_max(arr, m, n_pulls):
    def step(_, carry):
        cur, t = carry
        mi = jnp.max(cur, axis=1, keepdims=True)
        cur = jnp.where(cur >= mi, NEG_INF, cur)
        return cur, mi

    _, t = jax.lax.fori_loop(0, n_pulls, step, (arr, m), unroll=30)
    return t


def _attn_block_kernel(q_ref, k_ref, v_ref, attn_ref, ctx_ref, t_ref):
    qb = q_ref[0]
    kb = k_ref[0]
    s = jax.lax.dot_general(
        qb, kb, (((1,), (1,)), ((), ())), preferred_element_type=jnp.float32
    )
    S = s.shape[1]
    w = S // NSLICE

    m = jnp.max(s, axis=1, keepdims=True)

    sl = [s[:, i * w : (i + 1) * w] for i in range(NSLICE)]
    for i, j in _SORT_PAIRS:
        hi = jnp.maximum(sl[i], sl[j])
        lo = jnp.minimum(sl[i], sl[j])
        sl[i], sl[j] = hi, lo

    cand = jnp.concatenate(sl[:NCAND], axis=1)
    t_cand = _extract_kth_max(cand, m, TOPK)

    c_gt = jnp.sum((s > t_cand).astype(jnp.float32), axis=1, keepdims=True)
    ok = c_gt < TOPK
    t_ref[...] = t_cand

    @pl.when(jnp.logical_not(jnp.all(ok)))
    def _fallback():
        t_ref[...] = _extract_kth_max(s, m, TOPK)

    t = t_ref[...]
    wexp = jnp.where(s >= t, jnp.exp(s - m), 0.0)
    z = jnp.sum(wexp, axis=1, keepdims=True)
    attn = wexp / z
    attn_ref[0] = attn
    ctx_ref[0] = jax.lax.dot_general(
        attn, v_ref[0], (((1,), (0,)), ((), ())), preferred_element_type=jnp.float32
    )


@functools.partial(jax.jit, static_argnames=("interpret",))
def _run(q, k, v, interpret=False):
    bh, S, d = q.shape
    blk = min(512, S)
    grid = (bh, S // blk)
    attn, ctx = pl.pallas_call(
        _attn_block_kernel,
        grid=grid,
        in_specs=[
            pl.BlockSpec((1, blk, d), lambda h, i: (h, i, 0)),
            pl.BlockSpec((1, S, d), lambda h, i: (h, 0, 0)),
            pl.BlockSpec((1, S, d), lambda h, i: (h, 0, 0)),
        ],
        out_specs=[
            pl.BlockSpec((1, blk, S), lambda h, i: (h, i, 0)),
            pl.BlockSpec((1, blk, d), lambda h, i: (h, i, 0)),
        ],
        out_shape=[
            jax.ShapeDtypeStruct((bh, S, S), jnp.float32),
            jax.ShapeDtypeStruct((bh, S, d), jnp.float32),
        ],
        scratch_shapes=[pltpu.VMEM((blk, 1), jnp.float32)],
        compiler_params=pltpu.CompilerParams(
            dimension_semantics=("parallel", "arbitrary"),
        ),
        interpret=interpret,
    )(q, k, v)
    return ctx, attn


def kernel(q, k, v, B, num_heads):
    return _run(q, k, v)
